# baseline (speedup 1.0000x reference)
import jax
import jax.numpy as jnp
from jax.experimental import pallas as pl
from jax.experimental.pallas import tpu as pltpu

_BBLK = 1


def _exchange_body(w1_ref, w2_ref, t_ref, x0_ref, x1_ref, y1_ref, y2_ref):
    t = t_ref[0]
    m1 = jnp.abs(w1_ref[...]) >= t
    m2 = jnp.abs(w2_ref[...]) >= t
    a0 = x0_ref[...]
    a1 = x1_ref[...]
    y1_ref[...] = jnp.where(m1[None, None], a0, a1)
    y2_ref[...] = jnp.where(m2[None, None], a1, a0)


def kernel(x0, x1, bn1_weight, bn2_weight, bn_threshold):
    B, C, H, W = x0.shape
    x0t = jnp.transpose(x0, (0, 2, 3, 1))
    x1t = jnp.transpose(x1, (0, 2, 3, 1))
    w1 = bn1_weight.reshape(1, C)
    w2 = bn2_weight.reshape(1, C)
    t = bn_threshold.reshape(1)

    grid = (B // _BBLK,)
    blk = pl.BlockSpec((_BBLK, H, W, C), lambda b: (b, 0, 0, 0))
    wblk = pl.BlockSpec((1, C), lambda b: (0, 0))
    y1t, y2t = pl.pallas_call(
        _exchange_body,
        grid=grid,
        in_specs=[
            wblk,
            wblk,
            pl.BlockSpec(memory_space=pltpu.SMEM),
            blk,
            blk,
        ],
        out_specs=[blk, blk],
        out_shape=[jax.ShapeDtypeStruct((B, H, W, C), x0.dtype)] * 2,
    )(w1, w2, t, x0t, x1t)
    return (jnp.transpose(y1t, (0, 3, 1, 2)),
            jnp.transpose(y2t, (0, 3, 1, 2)))
